# baseline (speedup 1.0000x reference)
import functools

import jax
import jax.numpy as jnp
from jax import lax
from jax.experimental import pallas as pl
from jax.experimental.pallas import tpu as pltpu
from jax.experimental.pallas import tpu_sc as plsc

B = 16384
D = 64
HALF = B // 2
MARGIN = 1.0

_info = plsc.get_sparse_core_info()
NC, NS, L = _info.num_cores, _info.num_subcores, _info.num_lanes
NW = NC * NS
PPW = HALF // NW
CH = 128
NCH = (2 * PPW) // CH
GRP = CH // 16


def _sc_body(bh, bt, br, ent_re, ent_im, rel_re, rel_im,
             score_out, losspart_out,
             idx_h, idx_t, idx_r, ish_h, ish_t, ish_r,
             hre, him, tre, tim, rre, rim,
             score_v, loss_v, isem, gsem):
    w = lax.axis_index("s") * NC + lax.axis_index("c")
    pos_base = w * PPW
    neg_base = HALF + w * PPW

    bases = [pos_base + c * CH if c < NCH // 2 else neg_base + (c - NCH // 2) * CH
             for c in range(NCH)]

    copies = []
    for c in range(NCH):
        copies.append(pltpu.async_copy(bh.at[pl.ds(bases[c], CH)], idx_h.at[c], isem))
        copies.append(pltpu.async_copy(bt.at[pl.ds(bases[c], CH)], idx_t.at[c], isem))
        copies.append(pltpu.async_copy(br.at[pl.ds(bases[c], CH)], idx_r.at[c], isem))
    for cp in copies:
        cp.wait()

    for c in range(NCH):
        for j in range(CH // L):
            sl = pl.ds(j * L, L)
            ish_h[c, sl] = idx_h[c, sl] >> 1
            ish_t[c, sl] = idx_t[c, sl] >> 1
            ish_r[c, sl] = idx_r[c, sl] >> 1

    lane = lax.iota(jnp.int32, L)
    shuf = [lane ^ sh for sh in (8, 4, 2, 1)]

    def bcast(v, e):
        return v.at[jnp.full((L,), e, jnp.int32)].get(mode="promise_in_bounds")

    def hsum(v):
        for idx in shuf:
            v = v + v.at[idx].get(mode="promise_in_bounds")
        return v

    for c in range(NCH):
        gathers = [
            pltpu.async_copy(ent_re.at[ish_h.at[c]], hre, gsem),
            pltpu.async_copy(ent_im.at[ish_h.at[c]], him, gsem),
            pltpu.async_copy(ent_re.at[ish_t.at[c]], tre, gsem),
            pltpu.async_copy(ent_im.at[ish_t.at[c]], tim, gsem),
            pltpu.async_copy(rel_re.at[ish_r.at[c]], rre, gsem),
            pltpu.async_copy(rel_im.at[ish_r.at[c]], rim, gsem),
        ]
        for g in gathers:
            g.wait()

        def group_body(g, carry, c=c):
            gsl = pl.ds(g * L, L)
            parh = (idx_h[c, gsl] & 1).astype(jnp.float32)
            part = (idx_t[c, gsl] & 1).astype(jnp.float32)
            parr = (idx_r[c, gsl] & 1).astype(jnp.float32)
            scores = jnp.zeros((L,), jnp.float32)
            for e in range(16):
                row = g * L + e
                ph = bcast(parh, e)
                pt = bcast(part, e)
                pr = bcast(parr, e)
                acc = jnp.zeros((L,), jnp.float32)
                for q in range(D // L):
                    lo = pl.ds(q * L, L)
                    hi = pl.ds(D + q * L, L)
                    a_l = hre[row, lo]
                    a = a_l + ph * (hre[row, hi] - a_l)
                    b_l = him[row, lo]
                    b = b_l + ph * (him[row, hi] - b_l)
                    tr_l = tre[row, lo]
                    tr = tr_l + pt * (tre[row, hi] - tr_l)
                    ti_l = tim[row, lo]
                    ti = ti_l + pt * (tim[row, hi] - ti_l)
                    rr_l = rre[row, lo]
                    rr = rr_l + pr * (rre[row, hi] - rr_l)
                    ri_l = rim[row, lo]
                    ri = ri_l + pr * (rim[row, hi] - ri_l)
                    acc = acc + (a * tr + b * ti) * rr + (a * ti - b * tr) * ri
                scores = jnp.where(lane == e, hsum(acc), scores)
            score_v[pl.ds(c * CH + g * L, L)] = scores
            return carry

        lax.fori_loop(0, GRP, group_body, 0)

    pltpu.sync_copy(score_v.at[pl.ds(0, PPW)], score_out.at[pl.ds(pos_base, PPW)])
    pltpu.sync_copy(score_v.at[pl.ds(PPW, PPW)], score_out.at[pl.ds(neg_base, PPW)])

    acc = jnp.zeros((L,), jnp.float32)
    for j in range(PPW // L):
        p = score_v[pl.ds(j * L, L)]
        n = score_v[pl.ds(PPW + j * L, L)]
        acc = acc + jnp.maximum(0.0, p - n + MARGIN)
    loss_v[...] = acc
    pltpu.sync_copy(loss_v, losspart_out.at[w])


@functools.partial(
    pl.kernel,
    mesh=plsc.VectorSubcoreMesh(core_axis_name="c", subcore_axis_name="s"),
    compiler_params=pltpu.CompilerParams(use_tc_tiling_on_sc=True),
    out_type=[
        jax.ShapeDtypeStruct((B,), jnp.float32),
        jax.ShapeDtypeStruct((NW, L), jnp.float32),
    ],
    scratch_types=[
        pltpu.VMEM((NCH, CH), jnp.int32),
        pltpu.VMEM((NCH, CH), jnp.int32),
        pltpu.VMEM((NCH, CH), jnp.int32),
        pltpu.VMEM((NCH, CH), jnp.int32),
        pltpu.VMEM((NCH, CH), jnp.int32),
        pltpu.VMEM((NCH, CH), jnp.int32),
        pltpu.VMEM((CH, 2 * D), jnp.float32),
        pltpu.VMEM((CH, 2 * D), jnp.float32),
        pltpu.VMEM((CH, 2 * D), jnp.float32),
        pltpu.VMEM((CH, 2 * D), jnp.float32),
        pltpu.VMEM((CH, 2 * D), jnp.float32),
        pltpu.VMEM((CH, 2 * D), jnp.float32),
        pltpu.VMEM((2 * PPW,), jnp.float32),
        pltpu.VMEM((L,), jnp.float32),
        pltpu.SemaphoreType.DMA,
        pltpu.SemaphoreType.DMA,
    ],
)
def _sc_kernel(bh, bt, br, ent_re, ent_im, rel_re, rel_im, *rest):
    _sc_body(bh, bt, br, ent_re, ent_im, rel_re, rel_im, *rest)


def kernel(batch_h, batch_t, batch_r, batch_y, ent_re, ent_im, rel_re, rel_im):
    bh = batch_h.astype(jnp.int32)
    bt = batch_t.astype(jnp.int32)
    br = batch_r.astype(jnp.int32)
    ent_re2 = ent_re.reshape(ent_re.shape[0] // 2, 2 * D)
    ent_im2 = ent_im.reshape(ent_im.shape[0] // 2, 2 * D)
    rel_re2 = rel_re.reshape(rel_re.shape[0] // 2, 2 * D)
    rel_im2 = rel_im.reshape(rel_im.shape[0] // 2, 2 * D)
    score, losspart = _sc_kernel(bh, bt, br, ent_re2, ent_im2, rel_re2, rel_im2)
    loss = jnp.sum(losspart)
    return (loss, score[:HALF], score[HALF:], score)


# speedup vs baseline: 1.2522x; 1.2522x over previous
import functools

import jax
import jax.numpy as jnp
from jax import lax
from jax.experimental import pallas as pl
from jax.experimental.pallas import tpu as pltpu
from jax.experimental.pallas import tpu_sc as plsc

B = 16384
D = 64
HALF = B // 2
MARGIN = 1.0

_info = plsc.get_sparse_core_info()
NC, NS, L = _info.num_cores, _info.num_subcores, _info.num_lanes
NW = NC * NS
PPW = HALF // NW
CH = 128
NCH = (2 * PPW) // CH
GRP = CH // 16


def _sc_body(bh, bt, br, ent, rel,
             score_out, losspart_out,
             idx_h, idx_t, idx_r,
             hrow, trow, rrow,
             score_v, loss_v, isem, gsem):
    w = lax.axis_index("s") * NC + lax.axis_index("c")
    pos_base = w * PPW
    neg_base = HALF + w * PPW

    bases = [pos_base + c * CH if c < NCH // 2 else neg_base + (c - NCH // 2) * CH
             for c in range(NCH)]

    copies = []
    for c in range(NCH):
        copies.append(pltpu.async_copy(bh.at[pl.ds(bases[c], CH)], idx_h.at[c], isem))
        copies.append(pltpu.async_copy(bt.at[pl.ds(bases[c], CH)], idx_t.at[c], isem))
        copies.append(pltpu.async_copy(br.at[pl.ds(bases[c], CH)], idx_r.at[c], isem))
    for cp in copies:
        cp.wait()

    lane = lax.iota(jnp.int32, L)
    shuf = [lane ^ sh for sh in (8, 4, 2, 1)]

    def hsum(v):
        for idx in shuf:
            v = v + v.at[idx].get(mode="promise_in_bounds")
        return v

    for c in range(NCH):
        gathers = [
            pltpu.async_copy(ent.at[idx_h.at[c]], hrow, gsem),
            pltpu.async_copy(ent.at[idx_t.at[c]], trow, gsem),
            pltpu.async_copy(rel.at[idx_r.at[c]], rrow, gsem),
        ]
        for g in gathers:
            g.wait()

        def group_body(g, carry, c=c):
            scores = jnp.zeros((L,), jnp.float32)
            for e in range(16):
                row = g * L + e
                acc = jnp.zeros((L,), jnp.float32)
                for q in range(D // L):
                    re_s = pl.ds(q * L, L)
                    im_s = pl.ds(D + q * L, L)
                    a = hrow[row, re_s]
                    b = hrow[row, im_s]
                    tr = trow[row, re_s]
                    ti = trow[row, im_s]
                    rr = rrow[row, re_s]
                    ri = rrow[row, im_s]
                    acc = acc + (a * tr + b * ti) * rr + (a * ti - b * tr) * ri
                scores = jnp.where(lane == e, hsum(acc), scores)
            score_v[pl.ds(c * CH + g * L, L)] = scores
            return carry

        lax.fori_loop(0, GRP, group_body, 0)

    pltpu.sync_copy(score_v.at[pl.ds(0, PPW)], score_out.at[pl.ds(pos_base, PPW)])
    pltpu.sync_copy(score_v.at[pl.ds(PPW, PPW)], score_out.at[pl.ds(neg_base, PPW)])

    acc = jnp.zeros((L,), jnp.float32)
    for j in range(PPW // L):
        p = score_v[pl.ds(j * L, L)]
        n = score_v[pl.ds(PPW + j * L, L)]
        acc = acc + jnp.maximum(0.0, p - n + MARGIN)
    loss_v[...] = acc
    pltpu.sync_copy(loss_v, losspart_out.at[w])


@functools.partial(
    pl.kernel,
    mesh=plsc.VectorSubcoreMesh(core_axis_name="c", subcore_axis_name="s"),
    compiler_params=pltpu.CompilerParams(use_tc_tiling_on_sc=True),
    out_type=[
        jax.ShapeDtypeStruct((B,), jnp.float32),
        jax.ShapeDtypeStruct((NW, L), jnp.float32),
    ],
    scratch_types=[
        pltpu.VMEM((NCH, CH), jnp.int32),
        pltpu.VMEM((NCH, CH), jnp.int32),
        pltpu.VMEM((NCH, CH), jnp.int32),
        pltpu.VMEM((CH, 2 * D), jnp.float32),
        pltpu.VMEM((CH, 2 * D), jnp.float32),
        pltpu.VMEM((CH, 2 * D), jnp.float32),
        pltpu.VMEM((2 * PPW,), jnp.float32),
        pltpu.VMEM((L,), jnp.float32),
        pltpu.SemaphoreType.DMA,
        pltpu.SemaphoreType.DMA,
    ],
)
def _sc_kernel(bh, bt, br, ent, rel, *rest):
    _sc_body(bh, bt, br, ent, rel, *rest)


def kernel(batch_h, batch_t, batch_r, batch_y, ent_re, ent_im, rel_re, rel_im):
    bh = batch_h.astype(jnp.int32)
    bt = batch_t.astype(jnp.int32)
    br = batch_r.astype(jnp.int32)
    ent = jnp.concatenate([ent_re, ent_im], axis=1)
    rel = jnp.concatenate([rel_re, rel_im], axis=1)
    score, losspart = _sc_kernel(bh, bt, br, ent, rel)
    loss = jnp.sum(losspart)
    return (loss, score[:HALF], score[HALF:], score)


# speedup vs baseline: 2.3777x; 1.8988x over previous
import functools

import jax
import jax.numpy as jnp
from jax import lax
from jax.experimental import pallas as pl
from jax.experimental.pallas import tpu as pltpu
from jax.experimental.pallas import tpu_sc as plsc

B = 16384
D = 64
HALF = B // 2
MARGIN = 1.0
NENT = 1000000

_info = plsc.get_sparse_core_info()
NC, NS, L = _info.num_cores, _info.num_subcores, _info.num_lanes
NW = NC * NS
PPW = HALF // NW
CH = 128
NCH = (2 * PPW) // CH
GRP = CH // 16

TW = 8064


def _tc_cat_t_body(re_ref, im_ref, out_ref):
    eye = (lax.broadcasted_iota(jnp.int32, (D, D), 0)
           == lax.broadcasted_iota(jnp.int32, (D, D), 1)).astype(jnp.float32)
    dn = (((0,), (0,)), ((), ()))
    out_ref[:, 0:D] = lax.dot_general(
        re_ref[...], eye, dn, preferred_element_type=jnp.float32)
    out_ref[:, D:2 * D] = lax.dot_general(
        im_ref[...], eye, dn, preferred_element_type=jnp.float32)


_tc_cat_t = pl.pallas_call(
    _tc_cat_t_body,
    grid=((NENT + TW - 1) // TW,),
    in_specs=[
        pl.BlockSpec((D, TW), lambda i: (0, i)),
        pl.BlockSpec((D, TW), lambda i: (0, i)),
    ],
    out_specs=pl.BlockSpec((TW, 2 * D), lambda i: (i, 0)),
    out_shape=jax.ShapeDtypeStruct((NENT, 2 * D), jnp.float32),
)


def _sc_body(bh, bt, br, ent, rel,
             score_out, losspart_out,
             idx_h, idx_t, idx_r,
             hrow, trow, rrow,
             score_v, loss_v, isem, gsem):
    w = lax.axis_index("s") * NC + lax.axis_index("c")
    pos_base = w * PPW
    neg_base = HALF + w * PPW

    bases = [pos_base + c * CH if c < NCH // 2 else neg_base + (c - NCH // 2) * CH
             for c in range(NCH)]

    copies = []
    for c in range(NCH):
        copies.append(pltpu.async_copy(bh.at[pl.ds(bases[c], CH)], idx_h.at[c], isem))
        copies.append(pltpu.async_copy(bt.at[pl.ds(bases[c], CH)], idx_t.at[c], isem))
        copies.append(pltpu.async_copy(br.at[pl.ds(bases[c], CH)], idx_r.at[c], isem))
    for cp in copies:
        cp.wait()

    lane = lax.iota(jnp.int32, L)
    shuf = [lane ^ sh for sh in (8, 4, 2, 1)]

    def hsum(v):
        for idx in shuf:
            v = v + v.at[idx].get(mode="promise_in_bounds")
        return v

    for c in range(NCH):
        gathers = [
            pltpu.async_copy(ent.at[idx_h.at[c]], hrow, gsem),
            pltpu.async_copy(ent.at[idx_t.at[c]], trow, gsem),
            pltpu.async_copy(rel.at[idx_r.at[c]], rrow, gsem),
        ]
        for g in gathers:
            g.wait()

        def group_body(g, carry, c=c):
            scores = jnp.zeros((L,), jnp.float32)
            for e in range(16):
                row = g * L + e
                acc = jnp.zeros((L,), jnp.float32)
                for q in range(D // L):
                    re_s = pl.ds(q * L, L)
                    im_s = pl.ds(D + q * L, L)
                    a = hrow[row, re_s]
                    b = hrow[row, im_s]
                    tr = trow[row, re_s]
                    ti = trow[row, im_s]
                    rr = rrow[row, re_s]
                    ri = rrow[row, im_s]
                    acc = acc + (a * tr + b * ti) * rr + (a * ti - b * tr) * ri
                scores = jnp.where(lane == e, hsum(acc), scores)
            score_v[pl.ds(c * CH + g * L, L)] = scores
            return carry

        lax.fori_loop(0, GRP, group_body, 0)

    pltpu.sync_copy(score_v.at[pl.ds(0, PPW)], score_out.at[pl.ds(pos_base, PPW)])
    pltpu.sync_copy(score_v.at[pl.ds(PPW, PPW)], score_out.at[pl.ds(neg_base, PPW)])

    acc = jnp.zeros((L,), jnp.float32)
    for j in range(PPW // L):
        p = score_v[pl.ds(j * L, L)]
        n = score_v[pl.ds(PPW + j * L, L)]
        acc = acc + jnp.maximum(0.0, p - n + MARGIN)
    loss_v[...] = acc
    pltpu.sync_copy(loss_v, losspart_out.at[w])


@functools.partial(
    pl.kernel,
    mesh=plsc.VectorSubcoreMesh(core_axis_name="c", subcore_axis_name="s"),
    compiler_params=pltpu.CompilerParams(use_tc_tiling_on_sc=True),
    out_type=[
        jax.ShapeDtypeStruct((B,), jnp.float32),
        jax.ShapeDtypeStruct((NW, L), jnp.float32),
    ],
    scratch_types=[
        pltpu.VMEM((NCH, CH), jnp.int32),
        pltpu.VMEM((NCH, CH), jnp.int32),
        pltpu.VMEM((NCH, CH), jnp.int32),
        pltpu.VMEM((CH, 2 * D), jnp.float32),
        pltpu.VMEM((CH, 2 * D), jnp.float32),
        pltpu.VMEM((CH, 2 * D), jnp.float32),
        pltpu.VMEM((2 * PPW,), jnp.float32),
        pltpu.VMEM((L,), jnp.float32),
        pltpu.SemaphoreType.DMA,
        pltpu.SemaphoreType.DMA,
    ],
)
def _sc_kernel(bh, bt, br, ent, rel, *rest):
    _sc_body(bh, bt, br, ent, rel, *rest)


def kernel(batch_h, batch_t, batch_r, batch_y, ent_re, ent_im, rel_re, rel_im):
    bh = batch_h.astype(jnp.int32)
    bt = batch_t.astype(jnp.int32)
    br = batch_r.astype(jnp.int32)
    ent = _tc_cat_t(ent_re.T, ent_im.T)
    rel = jnp.concatenate([rel_re, rel_im], axis=1)
    score, losspart = _sc_kernel(bh, bt, br, ent, rel)
    loss = jnp.sum(losspart)
    return (loss, score[:HALF], score[HALF:], score)


# speedup vs baseline: 2.5331x; 1.0654x over previous
import functools

import jax
import jax.numpy as jnp
from jax import lax
from jax.experimental import pallas as pl
from jax.experimental.pallas import tpu as pltpu
from jax.experimental.pallas import tpu_sc as plsc

B = 16384
D = 64
HALF = B // 2
MARGIN = 1.0
NENT = 1000000

_info = plsc.get_sparse_core_info()
NC, NS, L = _info.num_cores, _info.num_subcores, _info.num_lanes
NW = NC * NS
PPW = HALF // NW
CH = 128
NCH = (2 * PPW) // CH
GRP = CH // 16

TW = 16128


def _tc_cat_t_body(re_ref, im_ref, out_ref):
    eye = (lax.broadcasted_iota(jnp.int32, (D, D), 0)
           == lax.broadcasted_iota(jnp.int32, (D, D), 1)).astype(jnp.float32)
    dn = (((0,), (0,)), ((), ()))
    out_ref[:, 0:D] = lax.dot_general(
        re_ref[...], eye, dn, preferred_element_type=jnp.float32)
    out_ref[:, D:2 * D] = lax.dot_general(
        im_ref[...], eye, dn, preferred_element_type=jnp.float32)


_tc_cat_t = pl.pallas_call(
    _tc_cat_t_body,
    grid=((NENT + TW - 1) // TW,),
    in_specs=[
        pl.BlockSpec((D, TW), lambda i: (0, i)),
        pl.BlockSpec((D, TW), lambda i: (0, i)),
    ],
    out_specs=pl.BlockSpec((TW, 2 * D), lambda i: (i, 0)),
    out_shape=jax.ShapeDtypeStruct((NENT, 2 * D), jnp.float32),
)


def _sc_body(bh, bt, br, ent, rel,
             score_out, losspart_out,
             idx_h, idx_t, idx_r,
             hrow, trow, rrow,
             score_v, loss_v, isem, gsem):
    w = lax.axis_index("s") * NC + lax.axis_index("c")
    pos_base = w * PPW
    neg_base = HALF + w * PPW

    bases = [pos_base + c * CH if c < NCH // 2 else neg_base + (c - NCH // 2) * CH
             for c in range(NCH)]

    copies = []
    for c in range(NCH):
        copies.append(pltpu.async_copy(bh.at[pl.ds(bases[c], CH)], idx_h.at[c], isem))
        copies.append(pltpu.async_copy(bt.at[pl.ds(bases[c], CH)], idx_t.at[c], isem))
        copies.append(pltpu.async_copy(br.at[pl.ds(bases[c], CH)], idx_r.at[c], isem))
    for cp in copies:
        cp.wait()

    lane = lax.iota(jnp.int32, L)
    shuf = [lane ^ sh for sh in (8, 4, 2, 1)]

    def hsum(v):
        for idx in shuf:
            v = v + v.at[idx].get(mode="promise_in_bounds")
        return v

    for c in range(NCH):
        gathers = [
            pltpu.async_copy(ent.at[idx_h.at[c]], hrow, gsem),
            pltpu.async_copy(ent.at[idx_t.at[c]], trow, gsem),
            pltpu.async_copy(rel.at[idx_r.at[c]], rrow, gsem),
        ]
        for g in gathers:
            g.wait()

        def group_body(g, carry, c=c):
            scores = jnp.zeros((L,), jnp.float32)
            for e in range(16):
                row = g * L + e
                acc = jnp.zeros((L,), jnp.float32)
                for q in range(D // L):
                    re_s = pl.ds(q * L, L)
                    im_s = pl.ds(D + q * L, L)
                    a = hrow[row, re_s]
                    b = hrow[row, im_s]
                    tr = trow[row, re_s]
                    ti = trow[row, im_s]
                    rr = rrow[row, re_s]
                    ri = rrow[row, im_s]
                    acc = acc + (a * tr + b * ti) * rr + (a * ti - b * tr) * ri
                scores = jnp.where(lane == e, hsum(acc), scores)
            score_v[pl.ds(c * CH + g * L, L)] = scores
            return carry

        lax.fori_loop(0, GRP, group_body, 0)

    pltpu.sync_copy(score_v.at[pl.ds(0, PPW)], score_out.at[pl.ds(pos_base, PPW)])
    pltpu.sync_copy(score_v.at[pl.ds(PPW, PPW)], score_out.at[pl.ds(neg_base, PPW)])

    acc = jnp.zeros((L,), jnp.float32)
    for j in range(PPW // L):
        p = score_v[pl.ds(j * L, L)]
        n = score_v[pl.ds(PPW + j * L, L)]
        acc = acc + jnp.maximum(0.0, p - n + MARGIN)
    loss_v[...] = acc
    pltpu.sync_copy(loss_v, losspart_out.at[w])


@functools.partial(
    pl.kernel,
    mesh=plsc.VectorSubcoreMesh(core_axis_name="c", subcore_axis_name="s"),
    compiler_params=pltpu.CompilerParams(use_tc_tiling_on_sc=True),
    out_type=[
        jax.ShapeDtypeStruct((B,), jnp.float32),
        jax.ShapeDtypeStruct((NW, L), jnp.float32),
    ],
    scratch_types=[
        pltpu.VMEM((NCH, CH), jnp.int32),
        pltpu.VMEM((NCH, CH), jnp.int32),
        pltpu.VMEM((NCH, CH), jnp.int32),
        pltpu.VMEM((CH, 2 * D), jnp.float32),
        pltpu.VMEM((CH, 2 * D), jnp.float32),
        pltpu.VMEM((CH, 2 * D), jnp.float32),
        pltpu.VMEM((2 * PPW,), jnp.float32),
        pltpu.VMEM((L,), jnp.float32),
        pltpu.SemaphoreType.DMA,
        pltpu.SemaphoreType.DMA,
    ],
)
def _sc_kernel(bh, bt, br, ent, rel, *rest):
    _sc_body(bh, bt, br, ent, rel, *rest)


def kernel(batch_h, batch_t, batch_r, batch_y, ent_re, ent_im, rel_re, rel_im):
    bh = batch_h.astype(jnp.int32)
    bt = batch_t.astype(jnp.int32)
    br = batch_r.astype(jnp.int32)
    ent = _tc_cat_t(ent_re.T, ent_im.T)
    rel = jnp.concatenate([rel_re, rel_im], axis=1)
    score, losspart = _sc_kernel(bh, bt, br, ent, rel)
    loss = jnp.sum(losspart)
    return (loss, score[:HALF], score[HALF:], score)


# speedup vs baseline: 2.5387x; 1.0022x over previous
import functools

import jax
import jax.numpy as jnp
from jax import lax
from jax.experimental import pallas as pl
from jax.experimental.pallas import tpu as pltpu
from jax.experimental.pallas import tpu_sc as plsc

B = 16384
D = 64
HALF = B // 2
MARGIN = 1.0
NENT = 1000000

_info = plsc.get_sparse_core_info()
NC, NS, L = _info.num_cores, _info.num_subcores, _info.num_lanes
NW = NC * NS
PPW = HALF // NW
CH = 128
NCH = (2 * PPW) // CH
GRP = CH // 16

TW = 24192


def _tc_cat_t_body(re_ref, im_ref, out_ref):
    eye = (lax.broadcasted_iota(jnp.int32, (D, D), 0)
           == lax.broadcasted_iota(jnp.int32, (D, D), 1)).astype(jnp.float32)
    dn = (((0,), (0,)), ((), ()))
    out_ref[:, 0:D] = lax.dot_general(
        re_ref[...], eye, dn, preferred_element_type=jnp.float32)
    out_ref[:, D:2 * D] = lax.dot_general(
        im_ref[...], eye, dn, preferred_element_type=jnp.float32)


_tc_cat_t = pl.pallas_call(
    _tc_cat_t_body,
    grid=((NENT + TW - 1) // TW,),
    in_specs=[
        pl.BlockSpec((D, TW), lambda i: (0, i)),
        pl.BlockSpec((D, TW), lambda i: (0, i)),
    ],
    out_specs=pl.BlockSpec((TW, 2 * D), lambda i: (i, 0)),
    out_shape=jax.ShapeDtypeStruct((NENT, 2 * D), jnp.float32),
)


def _sc_body(bh, bt, br, ent, rel,
             score_out, losspart_out,
             idx_h, idx_t, idx_r,
             hrow, trow, rrow,
             score_v, loss_v, isem, gsem):
    w = lax.axis_index("s") * NC + lax.axis_index("c")
    pos_base = w * PPW
    neg_base = HALF + w * PPW

    bases = [pos_base + c * CH if c < NCH // 2 else neg_base + (c - NCH // 2) * CH
             for c in range(NCH)]

    copies = []
    for c in range(NCH):
        copies.append(pltpu.async_copy(bh.at[pl.ds(bases[c], CH)], idx_h.at[c], isem))
        copies.append(pltpu.async_copy(bt.at[pl.ds(bases[c], CH)], idx_t.at[c], isem))
        copies.append(pltpu.async_copy(br.at[pl.ds(bases[c], CH)], idx_r.at[c], isem))
    for cp in copies:
        cp.wait()

    lane = lax.iota(jnp.int32, L)
    shuf = [lane ^ sh for sh in (8, 4, 2, 1)]

    def hsum(v):
        for idx in shuf:
            v = v + v.at[idx].get(mode="promise_in_bounds")
        return v

    for c in range(NCH):
        gathers = [
            pltpu.async_copy(ent.at[idx_h.at[c]], hrow, gsem),
            pltpu.async_copy(ent.at[idx_t.at[c]], trow, gsem),
            pltpu.async_copy(rel.at[idx_r.at[c]], rrow, gsem),
        ]
        for g in gathers:
            g.wait()

        def group_body(g, carry, c=c):
            scores = jnp.zeros((L,), jnp.float32)
            for e in range(16):
                row = g * L + e
                acc = jnp.zeros((L,), jnp.float32)
                for q in range(D // L):
                    re_s = pl.ds(q * L, L)
                    im_s = pl.ds(D + q * L, L)
                    a = hrow[row, re_s]
                    b = hrow[row, im_s]
                    tr = trow[row, re_s]
                    ti = trow[row, im_s]
                    rr = rrow[row, re_s]
                    ri = rrow[row, im_s]
                    acc = acc + (a * tr + b * ti) * rr + (a * ti - b * tr) * ri
                scores = jnp.where(lane == e, hsum(acc), scores)
            score_v[pl.ds(c * CH + g * L, L)] = scores
            return carry

        lax.fori_loop(0, GRP, group_body, 0)

    pltpu.sync_copy(score_v.at[pl.ds(0, PPW)], score_out.at[pl.ds(pos_base, PPW)])
    pltpu.sync_copy(score_v.at[pl.ds(PPW, PPW)], score_out.at[pl.ds(neg_base, PPW)])

    acc = jnp.zeros((L,), jnp.float32)
    for j in range(PPW // L):
        p = score_v[pl.ds(j * L, L)]
        n = score_v[pl.ds(PPW + j * L, L)]
        acc = acc + jnp.maximum(0.0, p - n + MARGIN)
    loss_v[...] = acc
    pltpu.sync_copy(loss_v, losspart_out.at[w])


@functools.partial(
    pl.kernel,
    mesh=plsc.VectorSubcoreMesh(core_axis_name="c", subcore_axis_name="s"),
    compiler_params=pltpu.CompilerParams(use_tc_tiling_on_sc=True),
    out_type=[
        jax.ShapeDtypeStruct((B,), jnp.float32),
        jax.ShapeDtypeStruct((NW, L), jnp.float32),
    ],
    scratch_types=[
        pltpu.VMEM((NCH, CH), jnp.int32),
        pltpu.VMEM((NCH, CH), jnp.int32),
        pltpu.VMEM((NCH, CH), jnp.int32),
        pltpu.VMEM((CH, 2 * D), jnp.float32),
        pltpu.VMEM((CH, 2 * D), jnp.float32),
        pltpu.VMEM((CH, 2 * D), jnp.float32),
        pltpu.VMEM((2 * PPW,), jnp.float32),
        pltpu.VMEM((L,), jnp.float32),
        pltpu.SemaphoreType.DMA,
        pltpu.SemaphoreType.DMA,
    ],
)
def _sc_kernel(bh, bt, br, ent, rel, *rest):
    _sc_body(bh, bt, br, ent, rel, *rest)


def kernel(batch_h, batch_t, batch_r, batch_y, ent_re, ent_im, rel_re, rel_im):
    bh = batch_h.astype(jnp.int32)
    bt = batch_t.astype(jnp.int32)
    br = batch_r.astype(jnp.int32)
    ent = _tc_cat_t(ent_re.T, ent_im.T)
    rel = jnp.concatenate([rel_re, rel_im], axis=1)
    score, losspart = _sc_kernel(bh, bt, br, ent, rel)
    loss = jnp.sum(losspart)
    return (loss, score[:HALF], score[HALF:], score)
